# baseline (speedup 1.0000x reference)
import jax
import jax.numpy as jnp
from jax import lax
from jax.experimental import pallas as pl
from jax.experimental.pallas import tpu as pltpu
from jax.experimental.pallas import tpu_sc as plsc

N_NODES = 10000
N_EDGES = 320000
D = 128
DH = D // 2

NC = 2
NS = 16
EPT = N_EDGES // NS
C = 160
NB = 5
CPB = EPT // C // NB
NR = 7
G = 6
NPAD = 10240
RPT = NPAD // NS
RCH = 32


def _sc_body(ncols_hbm, send_hbm, recv_hbm, out_hbm,
             sidx, ridx, rows, obuf, acc, gsems, isems):
    cid = lax.axis_index("c")
    sid = lax.axis_index("s")
    tbl = ncols_hbm.at[pl.ds(cid, 2 * N_NODES - 1)]

    pltpu.sync_copy(send_hbm.at[sid, 0], sidx.at[0])
    pltpu.sync_copy(recv_hbm.at[sid, 0], ridx.at[0])
    for t in range(G):
        pltpu.async_copy(tbl.at[sidx.at[0, t]], rows.at[t], gsems.at[t])

    def _zb(i, carry):
        obuf[i // (DH // 16), pl.ds((i % (DH // 16)) * 16, 16)] = (
            jnp.zeros((16,), jnp.float32))
        return carry
    lax.fori_loop(0, RCH * DH // 16, _zb, 0)
    r0 = sid * RPT
    for k in range(RPT // RCH):
        pltpu.async_copy(obuf, acc.at[pl.ds(r0 + k * RCH, RCH)], isems.at[0])
    for k in range(RPT // RCH):
        pltpu.make_async_copy(obuf, acc.at[pl.ds(r0 + k * RCH, RCH)],
                              isems.at[0]).wait()
    plsc.subcore_barrier()

    for b in range(NB):
        sl, nsl = b % 2, (b + 1) % 2
        off = (b * CPB) % NR

        if b + 1 < NB:
            pltpu.async_copy(send_hbm.at[sid, b + 1], sidx.at[nsl],
                             isems.at[0])
            pltpu.async_copy(recv_hbm.at[sid, b + 1], ridx.at[nsl],
                             isems.at[1])

        def _main(j, carry, sl=sl, off=off):
            jm = lax.rem(j + off, NR)
            jf = lax.rem(j + G + off, NR)
            pltpu.make_async_copy(tbl.at[sidx.at[sl, j]], rows.at[jm],
                                  gsems.at[jm]).wait()
            pltpu.sync_copy(rows.at[jm], acc.at[ridx.at[sl, j]], add=True)
            pltpu.async_copy(tbl.at[sidx.at[sl, j + G]], rows.at[jf],
                             gsems.at[jf])
            return carry
        lax.fori_loop(0, CPB - G, _main, 0)

        if b + 1 < NB:
            pltpu.make_async_copy(send_hbm.at[sid, b + 1], sidx.at[nsl],
                                  isems.at[0]).wait()
            pltpu.make_async_copy(recv_hbm.at[sid, b + 1], ridx.at[nsl],
                                  isems.at[1]).wait()

        def _tail(j, carry, sl=sl, nsl=nsl, off=off, last=(b + 1 == NB)):
            jm = lax.rem(j + off, NR)
            jf = lax.rem(j + G + off, NR)
            pltpu.make_async_copy(tbl.at[sidx.at[sl, j]], rows.at[jm],
                                  gsems.at[jm]).wait()
            pltpu.sync_copy(rows.at[jm], acc.at[ridx.at[sl, j]], add=True)
            if not last:
                pltpu.async_copy(tbl.at[sidx.at[nsl, j + G - CPB]],
                                 rows.at[jf], gsems.at[jf])
            return carry
        lax.fori_loop(CPB - G, CPB, _tail, 0)
    plsc.subcore_barrier()

    for k in range(RPT // RCH):
        pltpu.async_copy(acc.at[pl.ds(r0 + k * RCH, RCH)],
                         out_hbm.at[pl.ds(r0 + k * RCH, RCH)]
                         .at[:, pl.ds(cid * DH, DH)], isems.at[0])
    for k in range(RPT // RCH):
        pltpu.make_async_copy(acc.at[pl.ds(r0 + k * RCH, RCH)],
                              out_hbm.at[pl.ds(r0 + k * RCH, RCH)]
                              .at[:, pl.ds(cid * DH, DH)], isems.at[0]).wait()


_sc_aggregate = pl.kernel(
    _sc_body,
    out_type=jax.ShapeDtypeStruct((NPAD, D), jnp.float32),
    mesh=plsc.VectorSubcoreMesh(core_axis_name="c", subcore_axis_name="s",
                                num_cores=NC, num_subcores=NS),
    compiler_params=pltpu.CompilerParams(use_tc_tiling_on_sc=False),
    scratch_types=[
        pltpu.VMEM((2, CPB, C), jnp.int32),
        pltpu.VMEM((2, CPB, C), jnp.int32),
        pltpu.VMEM((NR, C, DH), jnp.float32),
        pltpu.VMEM((RCH, DH), jnp.float32),
        pltpu.VMEM_SHARED((NPAD, DH), jnp.float32),
        pltpu.SemaphoreType.DMA((NR,)),
        pltpu.SemaphoreType.DMA((2,)),
    ],
)


def _mlp_body(part_ref, nodes_ref, w1_ref, b1_ref, w2_ref, b2_ref, out_ref):
    h = part_ref[...] + nodes_ref[...]
    h1 = jnp.maximum(
        jnp.dot(h, w1_ref[...], preferred_element_type=jnp.float32)
        + b1_ref[...], 0.0)
    out_ref[...] = (jnp.dot(h1, w2_ref[...], preferred_element_type=jnp.float32)
                    + b2_ref[...])


_BLK = 5000


def _tc_mlp(partials, nodes, W1, b1, W2, b2):
    grid = N_NODES // _BLK
    return pl.pallas_call(
        _mlp_body,
        grid=(grid,),
        in_specs=[
            pl.BlockSpec((_BLK, D), lambda i: (i, 0)),
            pl.BlockSpec((_BLK, D), lambda i: (i, 0)),
            pl.BlockSpec((D, D), lambda i: (0, 0)),
            pl.BlockSpec((1, D), lambda i: (0, 0)),
            pl.BlockSpec((D, D), lambda i: (0, 0)),
            pl.BlockSpec((1, D), lambda i: (0, 0)),
        ],
        out_specs=pl.BlockSpec((_BLK, D), lambda i: (i, 0)),
        out_shape=jax.ShapeDtypeStruct((N_NODES, D), jnp.float32),
    )(partials, nodes, W1, b1, W2, b2)


def kernel(nodes, senders, receivers, W1, b1, W2, b2):
    ncols = nodes.reshape(2 * N_NODES, DH)
    send4d = (senders * 2).reshape(NS, NB, CPB, C)
    recv4d = receivers.reshape(NS, NB, CPB, C)
    partials = _sc_aggregate(ncols, send4d, recv4d)
    return _tc_mlp(partials, nodes, W1, b1.reshape(1, D), W2, b2.reshape(1, D))


# speedup vs baseline: 1.0041x; 1.0041x over previous
import jax
import jax.numpy as jnp
from jax import lax
from jax.experimental import pallas as pl
from jax.experimental.pallas import tpu as pltpu
from jax.experimental.pallas import tpu_sc as plsc

N_NODES = 10000
N_EDGES = 320000
D = 128
DH = D // 2

NC = 2
NS = 16
EPT = N_EDGES // NS
C = 160
NB = 5
CPB = EPT // C // NB
NR = 6
G = 5
NPAD = 10240
RPT = NPAD // NS
RCH = 64


def _sc_body(ncols_hbm, send_hbm, recv_hbm, out_hbm,
             sidx, ridx, rows, obuf, acc, gsems, isems):
    cid = lax.axis_index("c")
    sid = lax.axis_index("s")
    tbl = ncols_hbm.at[pl.ds(cid, 2 * N_NODES - 1)]

    pltpu.sync_copy(send_hbm.at[sid, 0], sidx.at[0])
    pltpu.sync_copy(recv_hbm.at[sid, 0], ridx.at[0])
    for t in range(G):
        pltpu.async_copy(tbl.at[sidx.at[0, t]], rows.at[t], gsems.at[t])

    def _zb(i, carry):
        obuf[i // (DH // 16), pl.ds((i % (DH // 16)) * 16, 16)] = (
            jnp.zeros((16,), jnp.float32))
        return carry
    lax.fori_loop(0, RCH * DH // 16, _zb, 0)
    r0 = sid * RPT
    for k in range(RPT // RCH):
        pltpu.async_copy(obuf, acc.at[pl.ds(r0 + k * RCH, RCH)], isems.at[0])
    for k in range(RPT // RCH):
        pltpu.make_async_copy(obuf, acc.at[pl.ds(r0 + k * RCH, RCH)],
                              isems.at[0]).wait()
    plsc.subcore_barrier()

    for b in range(NB):
        sl, nsl = b % 3, (b + 1) % 3
        off = (b * CPB) % NR

        if b + 1 < NB:
            pltpu.async_copy(send_hbm.at[sid, b + 1], sidx.at[nsl],
                             isems.at[0])
            pltpu.async_copy(recv_hbm.at[sid, b + 1], ridx.at[nsl],
                             isems.at[1])

        def _main(j, carry, sl=sl, off=off):
            jm = lax.rem(j + off, NR)
            jf = lax.rem(j + G + off, NR)
            pltpu.make_async_copy(tbl.at[sidx.at[sl, j]], rows.at[jm],
                                  gsems.at[jm]).wait()
            pltpu.sync_copy(rows.at[jm], acc.at[ridx.at[sl, j]], add=True)
            pltpu.async_copy(tbl.at[sidx.at[sl, j + G]], rows.at[jf],
                             gsems.at[jf])
            return carry
        lax.fori_loop(0, CPB - G, _main, 0)

        if b + 1 < NB:
            pltpu.make_async_copy(send_hbm.at[sid, b + 1], sidx.at[nsl],
                                  isems.at[0]).wait()
            pltpu.make_async_copy(recv_hbm.at[sid, b + 1], ridx.at[nsl],
                                  isems.at[1]).wait()

        def _tail(j, carry, sl=sl, nsl=nsl, off=off, last=(b + 1 == NB)):
            jm = lax.rem(j + off, NR)
            jf = lax.rem(j + G + off, NR)
            pltpu.make_async_copy(tbl.at[sidx.at[sl, j]], rows.at[jm],
                                  gsems.at[jm]).wait()
            pltpu.sync_copy(rows.at[jm], acc.at[ridx.at[sl, j]], add=True)
            if not last:
                pltpu.async_copy(tbl.at[sidx.at[nsl, j + G - CPB]],
                                 rows.at[jf], gsems.at[jf])
            return carry
        lax.fori_loop(CPB - G, CPB, _tail, 0)
    plsc.subcore_barrier()

    for k in range(RPT // RCH):
        pltpu.async_copy(acc.at[pl.ds(r0 + k * RCH, RCH)],
                         out_hbm.at[pl.ds(r0 + k * RCH, RCH)]
                         .at[:, pl.ds(cid * DH, DH)], isems.at[0])
    for k in range(RPT // RCH):
        pltpu.make_async_copy(acc.at[pl.ds(r0 + k * RCH, RCH)],
                              out_hbm.at[pl.ds(r0 + k * RCH, RCH)]
                              .at[:, pl.ds(cid * DH, DH)], isems.at[0]).wait()


_sc_aggregate = pl.kernel(
    _sc_body,
    out_type=jax.ShapeDtypeStruct((NPAD, D), jnp.float32),
    mesh=plsc.VectorSubcoreMesh(core_axis_name="c", subcore_axis_name="s",
                                num_cores=NC, num_subcores=NS),
    compiler_params=pltpu.CompilerParams(use_tc_tiling_on_sc=False),
    scratch_types=[
        pltpu.VMEM((3, CPB, C), jnp.int32),
        pltpu.VMEM((3, CPB, C), jnp.int32),
        pltpu.VMEM((NR, C, DH), jnp.float32),
        pltpu.VMEM((RCH, DH), jnp.float32),
        pltpu.VMEM_SHARED((NPAD, DH), jnp.float32),
        pltpu.SemaphoreType.DMA((NR,)),
        pltpu.SemaphoreType.DMA((2,)),
    ],
)


def _mlp_body(part_ref, nodes_ref, w1_ref, b1_ref, w2_ref, b2_ref, out_ref):
    h = part_ref[...] + nodes_ref[...]
    h1 = jnp.maximum(
        jnp.dot(h, w1_ref[...], preferred_element_type=jnp.float32)
        + b1_ref[...], 0.0)
    out_ref[...] = (jnp.dot(h1, w2_ref[...], preferred_element_type=jnp.float32)
                    + b2_ref[...])


_BLK = 5000


def _tc_mlp(partials, nodes, W1, b1, W2, b2):
    grid = N_NODES // _BLK
    return pl.pallas_call(
        _mlp_body,
        grid=(grid,),
        in_specs=[
            pl.BlockSpec((_BLK, D), lambda i: (i, 0)),
            pl.BlockSpec((_BLK, D), lambda i: (i, 0)),
            pl.BlockSpec((D, D), lambda i: (0, 0)),
            pl.BlockSpec((1, D), lambda i: (0, 0)),
            pl.BlockSpec((D, D), lambda i: (0, 0)),
            pl.BlockSpec((1, D), lambda i: (0, 0)),
        ],
        out_specs=pl.BlockSpec((_BLK, D), lambda i: (i, 0)),
        out_shape=jax.ShapeDtypeStruct((N_NODES, D), jnp.float32),
    )(partials, nodes, W1, b1, W2, b2)


def kernel(nodes, senders, receivers, W1, b1, W2, b2):
    ncols = nodes.reshape(2 * N_NODES, DH)
    send4d = (senders * 2).reshape(NS, NB, CPB, C)
    recv4d = receivers.reshape(NS, NB, CPB, C)
    partials = _sc_aggregate(ncols, send4d, recv4d)
    return _tc_mlp(partials, nodes, W1, b1.reshape(1, D), W2, b2.reshape(1, D))
